# baseline (speedup 1.0000x reference)
import functools

import jax
import jax.numpy as jnp
from jax.experimental import pallas as pl

_HID = 32
_FC = 256
_NR = 12
_CUTOFF = 6.0
_NMOL = 20

_BN = 1024
_BE = 4096
_BM = 8192


def _embed_body(hin_ref, w_ref, b_ref, o_ref):
    o_ref[...] = jax.nn.gelu(
        jnp.dot(hin_ref[...], w_ref[...], preferred_element_type=jnp.float32)
        + b_ref[...]
    )


def _mm_body(h_ref, w_ref, o_ref):
    o_ref[...] = jnp.dot(h_ref[...], w_ref[...],
                         preferred_element_type=jnp.float32)


def _edge_body(d_ref, msrc_ref, wr_ref, o_ref):
    d = d_ref[...]
    mu = jax.lax.broadcasted_iota(
        jnp.int32, (d.shape[0], _NR), 1).astype(jnp.float32) * (
        _CUTOFF / (_NR - 1))
    rbf = jnp.exp(-((d - mu) ** 2) * 2.0)
    fc = jnp.where(d < _CUTOFF,
                   0.5 * (jnp.cos(jnp.pi * d / _CUTOFF) + 1.0), 0.0)
    filt = jnp.dot(rbf, wr_ref[...], preferred_element_type=jnp.float32) * fc
    o_ref[...] = msrc_ref[...] * filt


def _update_body(h_ref, agg_ref, wu_ref, bu_ref, o_ref):
    o_ref[...] = h_ref[...] + jax.nn.gelu(
        jnp.dot(agg_ref[...], wu_ref[...], preferred_element_type=jnp.float32)
        + bu_ref[...]
    )


def _pool_body(h_ref, mol_ref, bat_ref, msk_ref, sum_ref, cnt_ref, mx_ref):
    @pl.when(pl.program_id(0) == 0)
    def _init():
        sum_ref[...] = jnp.zeros_like(sum_ref)
        cnt_ref[...] = jnp.zeros_like(cnt_ref)
        mx_ref[...] = jnp.full_like(mx_ref, -1e30)

    bat = bat_ref[...]
    msk = msk_ref[...]
    gid = jax.lax.broadcasted_iota(
        jnp.int32, (bat.shape[0], 50), 1).astype(jnp.float32)
    onehot = jnp.where((bat == gid) & (msk > 0.0), 1.0, 0.0)
    sum_ref[...] += jax.lax.dot_general(
        onehot, h_ref[...], (((0,), (0,)), ((), ())),
        preferred_element_type=jnp.float32)
    cnt_ref[...] += jnp.sum(onehot, axis=0, keepdims=True)
    mol = mol_ref[...]
    for k in range(_NMOL):
        mk = jnp.max(jnp.where(onehot > 0.0, mol[:, k][:, None], -1e30),
                     axis=0, keepdims=True)
        mx_ref[k:k + 1, :] = jnp.maximum(mx_ref[k:k + 1, :], mk)


def _head_body(sum_ref, cnt_ref, mx_ref, wp_ref, bp_ref, wmol_ref, bmol_ref,
               w0_ref, b0_ref, w1_ref, b1_ref, wo_ref, o_ref):
    cnt = cnt_ref[...]
    sums_p = jnp.dot(sum_ref[...], wp_ref[...],
                     preferred_element_type=jnp.float32) + cnt * bp_ref[...]
    pooled = sums_p / jnp.maximum(cnt, 1.0)
    mx = mx_ref[...]
    mx = jnp.where(mx > -1e29, mx, 0.0)
    mol_feats = jnp.dot(mx, wmol_ref[...],
                        preferred_element_type=jnp.float32) + bmol_ref[...]
    z = jnp.concatenate([pooled, mol_feats], axis=1)
    z = jax.nn.gelu(jnp.dot(z, w0_ref[...],
                            preferred_element_type=jnp.float32) + b0_ref[...])
    z = jax.nn.gelu(jnp.dot(z, w1_ref[...],
                            preferred_element_type=jnp.float32) + b1_ref[...])
    o_ref[...] = jnp.dot(z, wo_ref[...], preferred_element_type=jnp.float32)


def _row_kernel(body, n_rows, block, out_dim, *args):
    grid = n_rows // block
    in_specs = []
    ops = []
    for a, full in args:
        ops.append(a)
        if full:
            in_specs.append(
                pl.BlockSpec(a.shape, lambda i, nd=a.ndim: (0,) * nd))
        else:
            in_specs.append(
                pl.BlockSpec((block, a.shape[1]), lambda i: (i, 0)))
    return pl.pallas_call(
        body,
        grid=(grid,),
        in_specs=in_specs,
        out_specs=pl.BlockSpec((block, out_dim), lambda i: (i, 0)),
        out_shape=jax.ShapeDtypeStruct((n_rows, out_dim), jnp.float32),
    )(*ops)


def kernel(x, pos, edge_index, batch, W_embed, b_embed, Wm0, Wr0, Wu0, bu0,
           Wm1, Wr1, Wu1, bu1, W_proj, b_proj, W_mol, b_mol, W_fc0, b_fc0,
           W_fc1, b_fc1, W_out):
    n = x.shape[0]
    e = edge_index.shape[1]
    np_ = ((n + _BM - 1) // _BM) * _BM
    npool = ((n + _BN - 1) // _BN) * _BN
    ep = ((e + _BE - 1) // _BE) * _BE

    crystal = x[:, -1]
    h_in = jnp.concatenate([x[:, :30], crystal[:, None]], axis=1)
    h_in = jnp.pad(h_in, ((0, np_ - n), (0, 1)))
    We = jnp.pad(W_embed, ((0, 1), (0, 0)))
    b_embed2 = b_embed[None, :]

    src = edge_index[0]
    dst = edge_index[1]
    diff = pos[dst] - pos[src]
    d = jnp.sqrt(jnp.sum(diff * diff, axis=1) + 1e-12)
    d = jnp.pad(d, (0, ep - e), constant_values=100.0)[:, None]

    h = _row_kernel(_embed_body, np_, _BM, _HID,
                    (h_in, False), (We, True), (b_embed2, True))

    for Wm, Wr, Wu, bu in ((Wm0, Wr0, Wu0, bu0), (Wm1, Wr1, Wu1, bu1)):
        hm = _row_kernel(_mm_body, np_, _BM, _HID, (h, False), (Wm, True))
        msrc = jnp.pad(hm[src], ((0, ep - e), (0, 0)))
        msg = _row_kernel(_edge_body, ep, _BE, _HID,
                          (d, False), (msrc, False), (Wr, True))
        agg = jax.ops.segment_sum(msg[:e], dst, num_segments=n)
        agg = jnp.pad(agg, ((0, np_ - n), (0, 0)))
        h = _row_kernel(_update_body, np_, _BM, _HID,
                        (h, False), (agg, False), (Wu, True),
                        (bu[None, :], True))

    mask = (crystal > 0.5).astype(jnp.float32)
    mol_in = jnp.concatenate([x[:, 31:50], crystal[:, None]], axis=1)
    hp = jnp.pad(h[:n], ((0, npool - n), (0, 0)))
    molp = jnp.pad(mol_in, ((0, npool - n), (0, 0)))
    batf = jnp.pad(batch.astype(jnp.float32), (0, npool - n))[:, None]
    mskf = jnp.pad(mask, (0, npool - n))[:, None]

    sums, cnts, molmax = pl.pallas_call(
        _pool_body,
        grid=(npool // _BN,),
        in_specs=[
            pl.BlockSpec((_BN, _HID), lambda i: (i, 0)),
            pl.BlockSpec((_BN, _NMOL), lambda i: (i, 0)),
            pl.BlockSpec((_BN, 1), lambda i: (i, 0)),
            pl.BlockSpec((_BN, 1), lambda i: (i, 0)),
        ],
        out_specs=[
            pl.BlockSpec((50, _HID), lambda i: (0, 0)),
            pl.BlockSpec((1, 50), lambda i: (0, 0)),
            pl.BlockSpec((_NMOL, 50), lambda i: (0, 0)),
        ],
        out_shape=[
            jax.ShapeDtypeStruct((50, _HID), jnp.float32),
            jax.ShapeDtypeStruct((1, 50), jnp.float32),
            jax.ShapeDtypeStruct((_NMOL, 50), jnp.float32),
        ],
    )(hp, molp, batf, mskf)

    molmax = molmax.T
    cnt2 = cnts[0][:, None]
    small = [(sums, True), (cnt2, True), (molmax, True),
             (W_proj, True), (b_proj[None, :], True),
             (W_mol, True), (b_mol[None, :], True),
             (W_fc0, True), (b_fc0[None, :], True),
             (W_fc1, True), (b_fc1[None, :], True), (W_out, True)]
    out = pl.pallas_call(
        _head_body,
        in_specs=[pl.BlockSpec(a.shape, None) for a, _ in small],
        out_specs=pl.BlockSpec((50, 1), None),
        out_shape=jax.ShapeDtypeStruct((50, 1), jnp.float32),
    )(*[a for a, _ in small])
    return out
